# baseline (speedup 1.0000x reference)
import functools

import jax
import jax.numpy as jnp
from jax import lax
from jax.experimental import pallas as pl
from jax.experimental.pallas import tpu as pltpu
from jax.experimental.pallas import tpu_sc as plsc

N = 10000
E = 320000
D = 128

NC = 2
NS = 16
NW = NC * NS
EPW = E // NW
C = 128
NCH = -(-EPW // C)
CTAIL = EPW - (NCH - 1) * C
EPAD = C - CTAIL
IROWS = 80
WROWS = 8
NROWS = N + 8
WCH = 80
NZCH = N // WCH
KMAX = -(-NZCH // NS)


NCHP = IROWS
NPAIR = NCHP // 2


def _sc_body(node_hbm, src_hbm, dst_hbm, edge_hbm, out_hbm,
             dst_v, src_w, buf0, buf1, acc_sh, sem0, sem1, sem2):
    cid = lax.axis_index("c")
    sid = lax.axis_index("s")
    wid = sid * NC + cid

    pltpu.sync_copy(dst_hbm.at[wid], dst_v)

    def _zrow(r, _):
        for cc in range(D // 16):
            buf0[r, pl.ds(cc * 16, 16)] = jnp.zeros((16,), jnp.float32)
        return 0
    lax.fori_loop(0, WCH, _zrow, 0)

    def _zchunk(k, _):
        m = sid + k * NS

        @pl.when(m < NZCH)
        def _():
            pltpu.sync_copy(buf0.at[pl.ds(0, WCH)], acc_sh.at[pl.ds(m * WCH, WCH)])
        return 0
    lax.fori_loop(0, KMAX, _zchunk, 0)

    @pl.when(sid == 0)
    def _():
        pltpu.sync_copy(buf0.at[pl.ds(0, NROWS - N)], acc_sh.at[pl.ds(N, NROWS - N)])
    plsc.subcore_barrier()

    HC = C // 2

    def _issue_gather(r):
        pltpu.async_copy(node_hbm.at[src_w.at[r, pl.ds(0, HC)]],
                         buf0.at[pl.ds(0, HC)], sem0)
        pltpu.async_copy(node_hbm.at[src_w.at[r, pl.ds(HC, HC)]],
                         buf0.at[pl.ds(HC, HC)], sem2)

    pltpu.sync_copy(src_hbm.at[wid, pl.ds(0, WROWS)], src_w)
    _issue_gather(0)
    pltpu.async_copy(edge_hbm.at[wid, pl.ds(0, C)], buf1, sem1)

    def _pc(j, _):
        pltpu.make_async_copy(node_hbm.at[src_w.at[0, pl.ds(0, HC)]],
                              buf0.at[pl.ds(0, HC)], sem0).wait()
        pltpu.make_async_copy(node_hbm.at[src_w.at[0, pl.ds(0, HC)]],
                              buf0.at[pl.ds(HC, HC)], sem2).wait()
        pltpu.sync_copy(buf0, acc_sh.at[dst_v.at[j]], add=True)

        @pl.when(jnp.logical_and((j + 1) % WROWS == 0, j + 1 < NCH))
        def _():
            pltpu.sync_copy(
                src_hbm.at[wid, pl.ds(((j + 1) // WROWS) * WROWS, WROWS)], src_w)

        @pl.when(j + 1 < NCH)
        def _():
            _issue_gather((j + 1) % WROWS)

        @pl.when(j < NCH - 1)
        def _():
            pltpu.make_async_copy(edge_hbm.at[wid, pl.ds(0, C)], buf1, sem1).wait()

        @pl.when(j == NCH - 1)
        def _():
            pltpu.make_async_copy(edge_hbm.at[wid, pl.ds(0, CTAIL)],
                                  buf1.at[pl.ds(0, CTAIL)], sem1).wait()
        pltpu.sync_copy(buf1, acc_sh.at[dst_v.at[j]], add=True)

        nj = j + 1

        @pl.when(nj < NCH - 1)
        def _():
            pltpu.async_copy(edge_hbm.at[wid, pl.ds(nj * C, C)], buf1, sem1)

        @pl.when(nj == NCH - 1)
        def _():
            pltpu.async_copy(edge_hbm.at[wid, pl.ds(nj * C, CTAIL)],
                             buf1.at[pl.ds(0, CTAIL)], sem1)
        return 0
    lax.fori_loop(0, NCH, _pc, 0)
    plsc.subcore_barrier()

    def _wchunk(k, _):
        m = sid + k * NS

        @pl.when(m < NZCH)
        def _():
            pltpu.sync_copy(acc_sh.at[pl.ds(m * WCH, WCH)], buf0.at[pl.ds(0, WCH)])
            pltpu.sync_copy(buf0.at[pl.ds(0, WCH)], out_hbm.at[cid, pl.ds(m * WCH, WCH)])
        return 0
    lax.fori_loop(0, KMAX, _wchunk, 0)


@functools.lru_cache(maxsize=1)
def _sc_scatter():
    return functools.partial(
        pl.kernel,
        out_type=jax.ShapeDtypeStruct((NC, N, D), jnp.float32),
        mesh=plsc.VectorSubcoreMesh(core_axis_name="c", subcore_axis_name="s",
                                    num_cores=NC, num_subcores=NS),
        scratch_types=[
            pltpu.VMEM((IROWS, C), jnp.int32),
            pltpu.VMEM((WROWS, C), jnp.int32),
            pltpu.VMEM((C, D), jnp.float32),
            pltpu.VMEM((C, D), jnp.float32),
            pltpu.VMEM_SHARED((NROWS, D), jnp.float32),
            pltpu.SemaphoreType.DMA,
            pltpu.SemaphoreType.DMA,
            pltpu.SemaphoreType.DMA,
        ],
    )(_sc_body)


ROWS = 1000
NT = N // ROWS


def _mlp_pass1_body(eps_ref, x_ref, p0_ref, p1_ref, w1_ref, b1_ref, sums_ref):
    t = pl.program_id(0)
    rst = (1.0 + eps_ref[0]) * x_ref[...] + p0_ref[...] + p1_ref[...]
    h = jnp.dot(rst, w1_ref[...], preferred_element_type=jnp.float32)
    h = h + b1_ref[...]

    @pl.when(t == 0)
    def _():
        sums_ref[...] = jnp.zeros_like(sums_ref)

    sums_ref[0, :] += jnp.sum(h, axis=0)
    sums_ref[1, :] += jnp.sum(h * h, axis=0)


def _mlp_pass2_body(eps_ref, x_ref, p0_ref, p1_ref, w1_ref, b1_ref,
                    sums_ref, g_ref, be_ref, w2_ref, b2_ref, out_ref):
    rst = (1.0 + eps_ref[0]) * x_ref[...] + p0_ref[...] + p1_ref[...]
    h = jnp.dot(rst, w1_ref[...], preferred_element_type=jnp.float32)
    h = h + b1_ref[...]
    mean = sums_ref[0, :] / N
    var = sums_ref[1, :] / N - mean * mean
    hn = (h - mean[None, :]) * jax.lax.rsqrt(var + 1e-5)[None, :]
    hn = hn * g_ref[...] + be_ref[...]
    hn = jnp.maximum(hn, 0.0)
    out = jnp.dot(hn, w2_ref[...], preferred_element_type=jnp.float32)
    out_ref[...] = out + b2_ref[...]


def kernel(node_feat, edge_index, edge_feat, eps, W1, b1, gamma, beta, W2, b2):
    src2 = edge_index[0].reshape(NW, EPW)
    dst2 = edge_index[1].reshape(NW, EPW)
    pad_n = IROWS * C - EPW
    spread = jnp.tile(jnp.arange(8, dtype=jnp.int32) * 16, pad_n // 8 + 1)[:pad_n]
    src3 = jnp.concatenate(
        [src2, jnp.broadcast_to(spread, (NW, pad_n))],
        axis=1).reshape(NW, IROWS, C)
    dst3 = jnp.concatenate(
        [dst2, jnp.broadcast_to(N + spread // 16, (NW, pad_n))],
        axis=1).reshape(NW, IROWS, C)
    edge3 = edge_feat.reshape(NW, EPW, D)

    partials = _sc_scatter()(node_feat, src3, dst3, edge3)
    p0 = partials[0]
    p1 = partials[1]

    b1r = b1.reshape(1, 2 * D)
    b2r = b2.reshape(1, D)

    row_spec = pl.BlockSpec((ROWS, D), lambda t: (t, 0))
    full = lambda s: pl.BlockSpec(s, lambda t: tuple(0 for _ in s))
    eps_spec = pl.BlockSpec(memory_space=pltpu.SMEM)

    sums = pl.pallas_call(
        _mlp_pass1_body,
        grid=(NT,),
        in_specs=[eps_spec, row_spec, row_spec, row_spec,
                  full((D, 2 * D)), full((1, 2 * D))],
        out_specs=full((2, 2 * D)),
        out_shape=jax.ShapeDtypeStruct((2, 2 * D), jnp.float32),
    )(eps, node_feat, p0, p1, W1, b1r)

    out = pl.pallas_call(
        _mlp_pass2_body,
        grid=(NT,),
        in_specs=[eps_spec, row_spec, row_spec, row_spec,
                  full((D, 2 * D)), full((1, 2 * D)), full((2, 2 * D)),
                  full((1, 2 * D)), full((1, 2 * D)), full((2 * D, D)),
                  full((1, D))],
        out_specs=row_spec,
        out_shape=jax.ShapeDtypeStruct((N, D), jnp.float32),
    )(eps, node_feat, p0, p1, W1, b1r, sums, gamma.reshape(1, 2 * D),
      beta.reshape(1, 2 * D), W2, b2r)
    return out


# speedup vs baseline: 1.0008x; 1.0008x over previous
import functools

import jax
import jax.numpy as jnp
from jax import lax
from jax.experimental import pallas as pl
from jax.experimental.pallas import tpu as pltpu
from jax.experimental.pallas import tpu_sc as plsc

N = 10000
E = 320000
D = 128

NC = 2
NS = 16
NW = NC * NS
EPW = E // NW
C = 128
NCH = -(-EPW // C)
CTAIL = EPW - (NCH - 1) * C
EPAD = C - CTAIL
IROWS = 80
WROWS = 8
NROWS = N + 8
WCH = 80
NZCH = N // WCH
KMAX = -(-NZCH // NS)


NCHP = IROWS
NPAIR = NCHP // 2


def _sc_body(node_hbm, src_hbm, dst_hbm, edge_hbm, out_hbm,
             dst_v, src_w, buf0, buf1, acc_sh, sem0, sem1, sem2):
    cid = lax.axis_index("c")
    sid = lax.axis_index("s")
    wid = sid * NC + cid

    pltpu.sync_copy(dst_hbm.at[wid], dst_v)

    def _zrow(r, _):
        for cc in range(D // 16):
            buf0[r, pl.ds(cc * 16, 16)] = jnp.zeros((16,), jnp.float32)
        return 0
    lax.fori_loop(0, WCH, _zrow, 0)

    def _zchunk(k, _):
        m = sid + k * NS

        @pl.when(m < NZCH)
        def _():
            pltpu.sync_copy(buf0.at[pl.ds(0, WCH)], acc_sh.at[pl.ds(m * WCH, WCH)])
        return 0
    lax.fori_loop(0, KMAX, _zchunk, 0)

    @pl.when(sid == 0)
    def _():
        pltpu.sync_copy(buf0.at[pl.ds(0, NROWS - N)], acc_sh.at[pl.ds(N, NROWS - N)])
    plsc.subcore_barrier()

    HC = C // 2
    QC = C // 4

    def _issue_gather(r):
        pltpu.async_copy(node_hbm.at[src_w.at[r, pl.ds(0, QC)]],
                         buf0.at[pl.ds(0, QC)], sem0)
        pltpu.async_copy(node_hbm.at[src_w.at[r, pl.ds(QC, QC)]],
                         buf0.at[pl.ds(QC, QC)], sem0)
        pltpu.async_copy(node_hbm.at[src_w.at[r, pl.ds(HC, QC)]],
                         buf0.at[pl.ds(HC, QC)], sem2)
        pltpu.async_copy(node_hbm.at[src_w.at[r, pl.ds(HC + QC, QC)]],
                         buf0.at[pl.ds(HC + QC, QC)], sem2)

    pltpu.sync_copy(src_hbm.at[wid, pl.ds(0, WROWS)], src_w)
    _issue_gather(0)
    pltpu.async_copy(edge_hbm.at[wid, pl.ds(0, C)], buf1, sem1)

    def _pc(j, _):
        pltpu.make_async_copy(node_hbm.at[src_w.at[0, pl.ds(0, HC)]],
                              buf0.at[pl.ds(0, HC)], sem0).wait()
        pltpu.make_async_copy(node_hbm.at[src_w.at[0, pl.ds(0, HC)]],
                              buf0.at[pl.ds(HC, HC)], sem2).wait()
        pltpu.sync_copy(buf0, acc_sh.at[dst_v.at[j]], add=True)

        @pl.when(jnp.logical_and((j + 1) % WROWS == 0, j + 1 < NCH))
        def _():
            pltpu.sync_copy(
                src_hbm.at[wid, pl.ds(((j + 1) // WROWS) * WROWS, WROWS)], src_w)

        @pl.when(j + 1 < NCH)
        def _():
            _issue_gather((j + 1) % WROWS)

        @pl.when(j < NCH - 1)
        def _():
            pltpu.make_async_copy(edge_hbm.at[wid, pl.ds(0, C)], buf1, sem1).wait()

        @pl.when(j == NCH - 1)
        def _():
            pltpu.make_async_copy(edge_hbm.at[wid, pl.ds(0, CTAIL)],
                                  buf1.at[pl.ds(0, CTAIL)], sem1).wait()
        pltpu.sync_copy(buf1, acc_sh.at[dst_v.at[j]], add=True)

        nj = j + 1

        @pl.when(nj < NCH - 1)
        def _():
            pltpu.async_copy(edge_hbm.at[wid, pl.ds(nj * C, C)], buf1, sem1)

        @pl.when(nj == NCH - 1)
        def _():
            pltpu.async_copy(edge_hbm.at[wid, pl.ds(nj * C, CTAIL)],
                             buf1.at[pl.ds(0, CTAIL)], sem1)
        return 0
    lax.fori_loop(0, NCH, _pc, 0)
    plsc.subcore_barrier()

    def _wchunk(k, _):
        m = sid + k * NS

        @pl.when(m < NZCH)
        def _():
            pltpu.sync_copy(acc_sh.at[pl.ds(m * WCH, WCH)], buf0.at[pl.ds(0, WCH)])
            pltpu.sync_copy(buf0.at[pl.ds(0, WCH)], out_hbm.at[cid, pl.ds(m * WCH, WCH)])
        return 0
    lax.fori_loop(0, KMAX, _wchunk, 0)


@functools.lru_cache(maxsize=1)
def _sc_scatter():
    return functools.partial(
        pl.kernel,
        out_type=jax.ShapeDtypeStruct((NC, N, D), jnp.float32),
        mesh=plsc.VectorSubcoreMesh(core_axis_name="c", subcore_axis_name="s",
                                    num_cores=NC, num_subcores=NS),
        scratch_types=[
            pltpu.VMEM((IROWS, C), jnp.int32),
            pltpu.VMEM((WROWS, C), jnp.int32),
            pltpu.VMEM((C, D), jnp.float32),
            pltpu.VMEM((C, D), jnp.float32),
            pltpu.VMEM_SHARED((NROWS, D), jnp.float32),
            pltpu.SemaphoreType.DMA,
            pltpu.SemaphoreType.DMA,
            pltpu.SemaphoreType.DMA,
        ],
    )(_sc_body)


ROWS = 1000
NT = N // ROWS


def _mlp_pass1_body(eps_ref, x_ref, p0_ref, p1_ref, w1_ref, b1_ref, sums_ref):
    t = pl.program_id(0)
    rst = (1.0 + eps_ref[0]) * x_ref[...] + p0_ref[...] + p1_ref[...]
    h = jnp.dot(rst, w1_ref[...], preferred_element_type=jnp.float32)
    h = h + b1_ref[...]

    @pl.when(t == 0)
    def _():
        sums_ref[...] = jnp.zeros_like(sums_ref)

    sums_ref[0, :] += jnp.sum(h, axis=0)
    sums_ref[1, :] += jnp.sum(h * h, axis=0)


def _mlp_pass2_body(eps_ref, x_ref, p0_ref, p1_ref, w1_ref, b1_ref,
                    sums_ref, g_ref, be_ref, w2_ref, b2_ref, out_ref):
    rst = (1.0 + eps_ref[0]) * x_ref[...] + p0_ref[...] + p1_ref[...]
    h = jnp.dot(rst, w1_ref[...], preferred_element_type=jnp.float32)
    h = h + b1_ref[...]
    mean = sums_ref[0, :] / N
    var = sums_ref[1, :] / N - mean * mean
    hn = (h - mean[None, :]) * jax.lax.rsqrt(var + 1e-5)[None, :]
    hn = hn * g_ref[...] + be_ref[...]
    hn = jnp.maximum(hn, 0.0)
    out = jnp.dot(hn, w2_ref[...], preferred_element_type=jnp.float32)
    out_ref[...] = out + b2_ref[...]


def kernel(node_feat, edge_index, edge_feat, eps, W1, b1, gamma, beta, W2, b2):
    src2 = edge_index[0].reshape(NW, EPW)
    dst2 = edge_index[1].reshape(NW, EPW)
    pad_n = IROWS * C - EPW
    spread = jnp.tile(jnp.arange(8, dtype=jnp.int32) * 16, pad_n // 8 + 1)[:pad_n]
    src3 = jnp.concatenate(
        [src2, jnp.broadcast_to(spread, (NW, pad_n))],
        axis=1).reshape(NW, IROWS, C)
    dst3 = jnp.concatenate(
        [dst2, jnp.broadcast_to(N + spread // 16, (NW, pad_n))],
        axis=1).reshape(NW, IROWS, C)
    edge3 = edge_feat.reshape(NW, EPW, D)

    partials = _sc_scatter()(node_feat, src3, dst3, edge3)
    p0 = partials[0]
    p1 = partials[1]

    b1r = b1.reshape(1, 2 * D)
    b2r = b2.reshape(1, D)

    row_spec = pl.BlockSpec((ROWS, D), lambda t: (t, 0))
    full = lambda s: pl.BlockSpec(s, lambda t: tuple(0 for _ in s))
    eps_spec = pl.BlockSpec(memory_space=pltpu.SMEM)

    sums = pl.pallas_call(
        _mlp_pass1_body,
        grid=(NT,),
        in_specs=[eps_spec, row_spec, row_spec, row_spec,
                  full((D, 2 * D)), full((1, 2 * D))],
        out_specs=full((2, 2 * D)),
        out_shape=jax.ShapeDtypeStruct((2, 2 * D), jnp.float32),
    )(eps, node_feat, p0, p1, W1, b1r)

    out = pl.pallas_call(
        _mlp_pass2_body,
        grid=(NT,),
        in_specs=[eps_spec, row_spec, row_spec, row_spec,
                  full((D, 2 * D)), full((1, 2 * D)), full((2, 2 * D)),
                  full((1, 2 * D)), full((1, 2 * D)), full((2 * D, D)),
                  full((1, D))],
        out_specs=row_spec,
        out_shape=jax.ShapeDtypeStruct((N, D), jnp.float32),
    )(eps, node_feat, p0, p1, W1, b1r, sums, gamma.reshape(1, 2 * D),
      beta.reshape(1, 2 * D), W2, b2r)
    return out
